# baseline (speedup 1.0000x reference)
import jax
import jax.numpy as jnp
from jax.experimental import pallas as pl

T = 32768
D = 768
DH = D // 2
E = 64
K = 8
BT = 4096
CHT = 512

NEG_INF = float("-inf")


def _gate_kernel(xa_ref, xb_ref, wg_ref, idx_ref, w_ref):
    wg = wg_ref[...]
    subl = jax.lax.broadcasted_iota(jnp.int32, (E, CHT), 0).astype(jnp.float32)
    for c in range(BT // CHT):
        rows = pl.ds(c * CHT, CHT)
        xa = xa_ref[rows, :]
        xb = xb_ref[rows, :]
        lt = jax.lax.dot_general(
            wg[:, :DH], xa, (((1,), (1,)), ((), ())),
            preferred_element_type=jnp.float32)
        lt = lt + jax.lax.dot_general(
            wg[:, DH:], xb, (((1,), (1,)), ((), ())),
            preferred_element_type=jnp.float32)

        vals = []
        work = lt
        for k in range(K):
            m = jnp.max(work, axis=0, keepdims=True)
            vals.append(m)
            if k + 1 < K:
                work = jnp.where(work == m, NEG_INF, work)

        idxs = []
        for k in range(K):
            hit = lt == vals[k]
            idxs.append(jnp.min(jnp.where(hit, subl, jnp.float32(E)),
                                axis=0, keepdims=True))

        vt = jnp.concatenate(vals, axis=0)
        it = jnp.concatenate(idxs, axis=0)
        e = jnp.exp(vt - vt[0:1, :])
        w = e * (1.0 / jnp.sum(e, axis=0, keepdims=True))
        idx_ref[rows, :] = it.T.astype(jnp.int32)
        w_ref[rows, :] = w.T


@jax.jit
def kernel(hidden_states, wg_weight):
    wg = wg_weight.astype(jnp.float32)
    x = hidden_states.astype(jnp.float32)
    grid = (T // BT,)
    idx, w = pl.pallas_call(
        _gate_kernel,
        grid=grid,
        in_specs=[
            pl.BlockSpec((BT, DH), lambda i: (i, 0)),
            pl.BlockSpec((BT, DH), lambda i: (i, 1)),
            pl.BlockSpec((E, D), lambda i: (0, 0)),
        ],
        out_specs=[
            pl.BlockSpec((BT, K), lambda i: (i, 0)),
            pl.BlockSpec((BT, K), lambda i: (i, 0)),
        ],
        out_shape=[
            jax.ShapeDtypeStruct((T, K), jnp.int32),
            jax.ShapeDtypeStruct((T, K), jnp.float32),
        ],
    )(x, x, wg)
    return idx, w.astype(hidden_states.dtype)
